# Initial kernel scaffold; baseline (speedup 1.0000x reference)
import jax
import jax.numpy as jnp
from jax.experimental import pallas as pl


def kernel(x, edge_index_r0, edge_index_r1, edge_index_r2, edge_index_r3, Ws, bs):
    raise NotImplementedError("write your pallas kernel here")


# speedup vs baseline: 2.2419x; 2.2419x over previous
import functools

import jax
import jax.numpy as jnp
from jax import lax
from jax.experimental import pallas as pl
from jax.experimental.pallas import tpu as pltpu
from jax.experimental.pallas import tpu_sc as plsc

N = 10000
D = 128
R = 4
L = 4
E = 80000

NC = 2
NS = 16
LANES = 128

N_PAD = 10240
ACC_ROWS = N_PAD + 128
E_PAD = 81920
CHUNKS = E_PAD // LANES
CHUNKS_PER_SUB = CHUNKS // NS

ACC_PER_SUB = ACC_ROWS // NS
OUT_PER_SUB = N_PAD // NS
ZROWS = 24
ZREP = ACC_PER_SUB // ZROWS

_mesh = functools.partial(
    plsc.VectorSubcoreMesh, core_axis_name="c", subcore_axis_name="s"
)


def _scatter_kernel(t_hbm, pidx_hbm, parts_hbm,
                    rows_v, sidx_v, didx_v, zbuf, acc_sh, gsem):
  cid = lax.axis_index("c")
  sid = lax.axis_index("s")

  for i in range(ZROWS):
    for q in range(8):
      zbuf[i, pl.ds(q * 16, 16)] = jnp.zeros((16,), jnp.float32)

  for k in range(2):
    rel = cid * 2 + k
    for z in range(ZREP):
      pltpu.sync_copy(zbuf, acc_sh.at[pl.ds(sid * ACC_PER_SUB + z * ZROWS, ZROWS)])
    pltpu.sync_copy(
        pidx_hbm.at[rel, pl.ds(sid * CHUNKS_PER_SUB, CHUNKS_PER_SUB)], didx_v)

    for i in range(CHUNKS_PER_SUB):
      for q in range(8):
        v = didx_v[i, pl.ds(q * 16, 16)]
        src = jnp.minimum(v & 0xFFFF, N_PAD - 1) + rel * N_PAD
        sidx_v[i, pl.ds(q * 16, 16)] = src
        didx_v[i, pl.ds(q * 16, 16)] = v >> 16

    plsc.subcore_barrier()

    @pl.loop(0, CHUNKS_PER_SUB)
    def _(j):
      pltpu.async_copy(t_hbm.at[sidx_v.at[j]], rows_v, gsem).wait()
      pltpu.sync_copy(rows_v, acc_sh.at[didx_v.at[j]], add=True)

    plsc.subcore_barrier()
    pltpu.sync_copy(
        acc_sh.at[pl.ds(sid * OUT_PER_SUB, OUT_PER_SUB)],
        parts_hbm.at[rel, pl.ds(sid * OUT_PER_SUB, OUT_PER_SUB)],
    )
    plsc.subcore_barrier()


def _sc_scatter(t, pidx):
  return pl.kernel(
      _scatter_kernel,
      out_type=jax.ShapeDtypeStruct((R, N_PAD, D), jnp.float32),
      mesh=_mesh(),
      scratch_types=[
          pltpu.VMEM((LANES, D), jnp.float32),
          pltpu.VMEM((CHUNKS_PER_SUB, LANES), jnp.int32),
          pltpu.VMEM((CHUNKS_PER_SUB, LANES), jnp.int32),
          pltpu.VMEM((ZROWS, D), jnp.float32),
          pltpu.VMEM_SHARED((ACC_ROWS, D), jnp.float32),
          pltpu.SemaphoreType.DMA,
      ],
  )(t, pidx)


BN = 1024


def _tc_prep_body(pout, pin, x, w, t0, ins, outs):
  for r in range(R):
    od = pout[r, :, 0]
    idg = pin[r, :, 0]
    os_ = lax.rsqrt(jnp.maximum(od, 1.0))
    is_ = lax.rsqrt(jnp.maximum(idg, 1.0))
    outs[r, :] = os_
    ins[r, :] = is_
    t0[r, :, :] = jnp.dot(os_[:, None] * x[...], w[r],
                          preferred_element_type=jnp.float32)


def _tc_prep(pout, pin, x_pad, w0):
  grid = (N_PAD // BN,)
  return pl.pallas_call(
      _tc_prep_body,
      grid=grid,
      in_specs=[
          pl.BlockSpec((R, BN, D), lambda i: (0, i, 0)),
          pl.BlockSpec((R, BN, D), lambda i: (0, i, 0)),
          pl.BlockSpec((BN, D), lambda i: (i, 0)),
          pl.BlockSpec((R, D, D), lambda i: (0, 0, 0)),
      ],
      out_specs=[
          pl.BlockSpec((R, BN, D), lambda i: (0, i, 0)),
          pl.BlockSpec((R, BN), lambda i: (0, i)),
          pl.BlockSpec((R, BN), lambda i: (0, i)),
      ],
      out_shape=[
          jax.ShapeDtypeStruct((R, N_PAD, D), jnp.float32),
          jax.ShapeDtypeStruct((R, N_PAD), jnp.float32),
          jax.ShapeDtypeStruct((R, N_PAD), jnp.float32),
      ],
  )(pout, pin, x_pad, w0)


def _tc_mid_body(parts, ins, outs, wn, b, tn):
  h = jnp.zeros_like(parts[0])
  for r in range(R):
    h = h + jnp.maximum(ins[r, :][:, None] * parts[r] + b[0, r][None, :], 0.0)
  for r in range(R):
    tn[r, :, :] = jnp.dot(outs[r, :][:, None] * h, wn[r],
                          preferred_element_type=jnp.float32)


def _tc_mid(parts, ins, outs, wn, b_l):
  grid = (N_PAD // BN,)
  return pl.pallas_call(
      _tc_mid_body,
      grid=grid,
      in_specs=[
          pl.BlockSpec((R, BN, D), lambda i: (0, i, 0)),
          pl.BlockSpec((R, BN), lambda i: (0, i)),
          pl.BlockSpec((R, BN), lambda i: (0, i)),
          pl.BlockSpec((R, D, D), lambda i: (0, 0, 0)),
          pl.BlockSpec((1, R, D), lambda i: (0, 0, 0)),
      ],
      out_specs=pl.BlockSpec((R, BN, D), lambda i: (0, i, 0)),
      out_shape=jax.ShapeDtypeStruct((R, N_PAD, D), jnp.float32),
  )(parts, ins, outs, wn, b_l)


def _tc_final_body(parts, ins, b, h_out):
  h = b[0].sum(axis=0)[None, :]
  for r in range(R):
    h = h + ins[r, :][:, None] * parts[r]
  h_out[...] = h


def _tc_final(parts, ins, b_l):
  grid = (N_PAD // BN,)
  return pl.pallas_call(
      _tc_final_body,
      grid=grid,
      in_specs=[
          pl.BlockSpec((R, BN, D), lambda i: (0, i, 0)),
          pl.BlockSpec((R, BN), lambda i: (0, i)),
          pl.BlockSpec((1, R, D), lambda i: (0, 0, 0)),
      ],
      out_specs=pl.BlockSpec((BN, D), lambda i: (i, 0)),
      out_shape=jax.ShapeDtypeStruct((N_PAD, D), jnp.float32),
  )(parts, ins, b_l)


def kernel(x, edge_index_r0, edge_index_r1, edge_index_r2, edge_index_r3, Ws, bs):
  edges = [edge_index_r0, edge_index_r1, edge_index_r2, edge_index_r3]

  pad = E_PAD - E
  pidx, pidx_out = [], []
  for r in range(R):
    src = jnp.concatenate([edges[r][0], jnp.full((pad,), N_PAD, jnp.int32)])
    dst = jnp.concatenate([edges[r][1], jnp.full((pad,), N_PAD, jnp.int32)])
    pidx.append(((dst << 16) | src).reshape(CHUNKS, LANES))
    pidx_out.append(((src << 16) | src).reshape(CHUNKS, LANES))
  pidx = jnp.stack(pidx)
  pidx_out = jnp.stack(pidx_out)

  x_pad = jnp.pad(x, ((0, N_PAD - N), (0, 0)))

  ones_t = jnp.ones((R * N_PAD, D), jnp.float32)
  pin = _sc_scatter(ones_t, pidx)
  pout = _sc_scatter(ones_t, pidx_out)

  t, ins, outs = _tc_prep(pout, pin, x_pad, Ws[0])
  for l in range(L):
    parts = _sc_scatter(t.reshape(R * N_PAD, D), pidx)
    b_l = bs[l][None]
    if l < L - 1:
      t = _tc_mid(parts, ins, outs, Ws[l + 1], b_l)
    else:
      h = _tc_final(parts, ins, b_l)
  return h[:N]


# speedup vs baseline: 2.5136x; 1.1212x over previous
import functools

import jax
import jax.numpy as jnp
from jax import lax
from jax.experimental import pallas as pl
from jax.experimental.pallas import tpu as pltpu
from jax.experimental.pallas import tpu_sc as plsc

N = 10000
D = 128
R = 4
L = 4
E = 80000

NC = 2
NS = 16
LANES = 128

N_PAD = 10240
ACC_ROWS = N_PAD + 128
E_PAD = 81920
CHUNKS = E_PAD // LANES
CHUNKS_PER_SUB = CHUNKS // NS

ACC_PER_SUB = ACC_ROWS // NS
OUT_PER_SUB = N_PAD // NS
ZROWS = 24
ZREP = ACC_PER_SUB // ZROWS

_mesh = functools.partial(
    plsc.VectorSubcoreMesh, core_axis_name="c", subcore_axis_name="s"
)


def _scatter_kernel(t_hbm, pidx_hbm, parts_hbm,
                    rows0_v, rows1_v, sidx_v, didx_v, zbuf, acc_sh,
                    gsem0, gsem1):
  cid = lax.axis_index("c")
  sid = lax.axis_index("s")

  for i in range(ZROWS):
    for q in range(8):
      zbuf[i, pl.ds(q * 16, 16)] = jnp.zeros((16,), jnp.float32)

  for k in range(2):
    rel = cid * 2 + k
    for z in range(ZREP):
      pltpu.sync_copy(zbuf, acc_sh.at[pl.ds(sid * ACC_PER_SUB + z * ZROWS, ZROWS)])
    pltpu.sync_copy(
        pidx_hbm.at[rel, pl.ds(sid * CHUNKS_PER_SUB, CHUNKS_PER_SUB)], didx_v)

    for i in range(CHUNKS_PER_SUB):
      for q in range(8):
        v = didx_v[i, pl.ds(q * 16, 16)]
        src = jnp.minimum(v & 0xFFFF, N_PAD - 1) + rel * N_PAD
        sidx_v[i, pl.ds(q * 16, 16)] = src
        didx_v[i, pl.ds(q * 16, 16)] = v >> 16

    plsc.subcore_barrier()

    rows = (rows0_v, rows1_v)
    sems = (gsem0, gsem1)
    pend = pltpu.async_copy(t_hbm.at[sidx_v.at[0]], rows[0], sems[0])
    for j in range(CHUNKS_PER_SUB):
      pend.wait()
      if j + 1 < CHUNKS_PER_SUB:
        pend = pltpu.async_copy(
            t_hbm.at[sidx_v.at[j + 1]], rows[(j + 1) % 2], sems[(j + 1) % 2])
      pltpu.sync_copy(rows[j % 2], acc_sh.at[didx_v.at[j]], add=True)

    plsc.subcore_barrier()
    pltpu.sync_copy(
        acc_sh.at[pl.ds(sid * OUT_PER_SUB, OUT_PER_SUB)],
        parts_hbm.at[rel, pl.ds(sid * OUT_PER_SUB, OUT_PER_SUB)],
    )
    plsc.subcore_barrier()


def _sc_scatter(t, pidx):
  return pl.kernel(
      _scatter_kernel,
      out_type=jax.ShapeDtypeStruct((R, N_PAD, D), jnp.float32),
      mesh=_mesh(),
      scratch_types=[
          pltpu.VMEM((LANES, D), jnp.float32),
          pltpu.VMEM((LANES, D), jnp.float32),
          pltpu.VMEM((CHUNKS_PER_SUB, LANES), jnp.int32),
          pltpu.VMEM((CHUNKS_PER_SUB, LANES), jnp.int32),
          pltpu.VMEM((ZROWS, D), jnp.float32),
          pltpu.VMEM_SHARED((ACC_ROWS, D), jnp.float32),
          pltpu.SemaphoreType.DMA,
          pltpu.SemaphoreType.DMA,
      ],
  )(t, pidx)


BN = 1024


def _tc_prep_body(pout, pin, x, w, t0, ins, outs):
  for r in range(R):
    od = pout[r, :, 0]
    idg = pin[r, :, 0]
    os_ = lax.rsqrt(jnp.maximum(od, 1.0))
    is_ = lax.rsqrt(jnp.maximum(idg, 1.0))
    outs[r, :] = os_
    ins[r, :] = is_
    t0[r, :, :] = jnp.dot(os_[:, None] * x[...], w[r],
                          preferred_element_type=jnp.float32)


def _tc_prep(pout, pin, x_pad, w0):
  grid = (N_PAD // BN,)
  return pl.pallas_call(
      _tc_prep_body,
      grid=grid,
      in_specs=[
          pl.BlockSpec((R, BN, D), lambda i: (0, i, 0)),
          pl.BlockSpec((R, BN, D), lambda i: (0, i, 0)),
          pl.BlockSpec((BN, D), lambda i: (i, 0)),
          pl.BlockSpec((R, D, D), lambda i: (0, 0, 0)),
      ],
      out_specs=[
          pl.BlockSpec((R, BN, D), lambda i: (0, i, 0)),
          pl.BlockSpec((R, BN), lambda i: (0, i)),
          pl.BlockSpec((R, BN), lambda i: (0, i)),
      ],
      out_shape=[
          jax.ShapeDtypeStruct((R, N_PAD, D), jnp.float32),
          jax.ShapeDtypeStruct((R, N_PAD), jnp.float32),
          jax.ShapeDtypeStruct((R, N_PAD), jnp.float32),
      ],
  )(pout, pin, x_pad, w0)


def _tc_mid_body(parts, ins, outs, wn, b, tn):
  h = jnp.zeros_like(parts[0])
  for r in range(R):
    h = h + jnp.maximum(ins[r, :][:, None] * parts[r] + b[0, r][None, :], 0.0)
  for r in range(R):
    tn[r, :, :] = jnp.dot(outs[r, :][:, None] * h, wn[r],
                          preferred_element_type=jnp.float32)


def _tc_mid(parts, ins, outs, wn, b_l):
  grid = (N_PAD // BN,)
  return pl.pallas_call(
      _tc_mid_body,
      grid=grid,
      in_specs=[
          pl.BlockSpec((R, BN, D), lambda i: (0, i, 0)),
          pl.BlockSpec((R, BN), lambda i: (0, i)),
          pl.BlockSpec((R, BN), lambda i: (0, i)),
          pl.BlockSpec((R, D, D), lambda i: (0, 0, 0)),
          pl.BlockSpec((1, R, D), lambda i: (0, 0, 0)),
      ],
      out_specs=pl.BlockSpec((R, BN, D), lambda i: (0, i, 0)),
      out_shape=jax.ShapeDtypeStruct((R, N_PAD, D), jnp.float32),
  )(parts, ins, outs, wn, b_l)


def _tc_final_body(parts, ins, b, h_out):
  h = b[0].sum(axis=0)[None, :]
  for r in range(R):
    h = h + ins[r, :][:, None] * parts[r]
  h_out[...] = h


def _tc_final(parts, ins, b_l):
  grid = (N_PAD // BN,)
  return pl.pallas_call(
      _tc_final_body,
      grid=grid,
      in_specs=[
          pl.BlockSpec((R, BN, D), lambda i: (0, i, 0)),
          pl.BlockSpec((R, BN), lambda i: (0, i)),
          pl.BlockSpec((1, R, D), lambda i: (0, 0, 0)),
      ],
      out_specs=pl.BlockSpec((BN, D), lambda i: (i, 0)),
      out_shape=jax.ShapeDtypeStruct((N_PAD, D), jnp.float32),
  )(parts, ins, b_l)


def kernel(x, edge_index_r0, edge_index_r1, edge_index_r2, edge_index_r3, Ws, bs):
  edges = [edge_index_r0, edge_index_r1, edge_index_r2, edge_index_r3]

  pad = E_PAD - E
  pidx, pidx_out = [], []
  for r in range(R):
    src = jnp.concatenate([edges[r][0], jnp.full((pad,), N_PAD, jnp.int32)])
    dst = jnp.concatenate([edges[r][1], jnp.full((pad,), N_PAD, jnp.int32)])
    pidx.append(((dst << 16) | src).reshape(CHUNKS, LANES))
    pidx_out.append(((src << 16) | src).reshape(CHUNKS, LANES))
  pidx = jnp.stack(pidx)
  pidx_out = jnp.stack(pidx_out)

  x_pad = jnp.pad(x, ((0, N_PAD - N), (0, 0)))

  ones_t = jnp.ones((R * N_PAD, D), jnp.float32)
  pin = _sc_scatter(ones_t, pidx)
  pout = _sc_scatter(ones_t, pidx_out)

  t, ins, outs = _tc_prep(pout, pin, x_pad, Ws[0])
  for l in range(L):
    parts = _sc_scatter(t.reshape(R * N_PAD, D), pidx)
    b_l = bs[l][None]
    if l < L - 1:
      t = _tc_mid(parts, ins, outs, Ws[l + 1], b_l)
    else:
      h = _tc_final(parts, ins, b_l)
  return h[:N]


# speedup vs baseline: 3.1919x; 1.2699x over previous
import functools

import jax
import jax.numpy as jnp
from jax import lax
from jax.experimental import pallas as pl
from jax.experimental.pallas import tpu as pltpu
from jax.experimental.pallas import tpu_sc as plsc

N = 10000
D = 128
R = 4
L = 4
E = 80000

NC = 2
NS = 16
LANES = 128

N_PAD = 10240
ACC_ROWS = N_PAD + 128
E_PAD = 81920
CHUNKS = E_PAD // LANES
CHUNKS_PER_SUB = CHUNKS // NS

ACC_PER_SUB = ACC_ROWS // NS
OUT_PER_SUB = N_PAD // NS
ZROWS = 24
ZREP = ACC_PER_SUB // ZROWS

_mesh = functools.partial(
    plsc.VectorSubcoreMesh, core_axis_name="c", subcore_axis_name="s"
)


def _count_kernel(pidx_hbm, degs_hbm, ones_v, sidx_v, didx_v, zbuf, acc_sh):
  cid = lax.axis_index("c")
  sid = lax.axis_index("s")

  for i in range(ZROWS):
    for q in range(8):
      zbuf[i, pl.ds(q * 16, 16)] = jnp.zeros((16,), jnp.float32)
  for i in range(LANES):
    for q in range(8):
      ones_v[i, pl.ds(q * 16, 16)] = jnp.ones((16,), jnp.float32)

  for k in range(2):
    rel = cid * 2 + k
    pltpu.sync_copy(
        pidx_hbm.at[rel, pl.ds(sid * CHUNKS_PER_SUB, CHUNKS_PER_SUB)], didx_v)
    for i in range(CHUNKS_PER_SUB):
      for q in range(8):
        v = didx_v[i, pl.ds(q * 16, 16)]
        sidx_v[i, pl.ds(q * 16, 16)] = v & 0xFFFF
        didx_v[i, pl.ds(q * 16, 16)] = v >> 16

    for d in range(2):
      idx_v = sidx_v if d == 0 else didx_v
      pp = rel * 2 + d
      for z in range(ZREP):
        pltpu.sync_copy(zbuf, acc_sh.at[pl.ds(sid * ACC_PER_SUB + z * ZROWS, ZROWS)])
      plsc.subcore_barrier()

      @pl.loop(0, CHUNKS_PER_SUB)
      def _(j):
        pltpu.sync_copy(ones_v, acc_sh.at[idx_v.at[j]], add=True)

      plsc.subcore_barrier()
      pltpu.sync_copy(
          acc_sh.at[pl.ds(sid * OUT_PER_SUB, OUT_PER_SUB)],
          degs_hbm.at[pp, pl.ds(sid * OUT_PER_SUB, OUT_PER_SUB)],
      )
      plsc.subcore_barrier()


def _sc_count(pidx):
  return pl.kernel(
      _count_kernel,
      out_type=jax.ShapeDtypeStruct((2 * R, N_PAD, D), jnp.float32),
      mesh=_mesh(),
      scratch_types=[
          pltpu.VMEM((LANES, D), jnp.float32),
          pltpu.VMEM((CHUNKS_PER_SUB, LANES), jnp.int32),
          pltpu.VMEM((CHUNKS_PER_SUB, LANES), jnp.int32),
          pltpu.VMEM((ZROWS, D), jnp.float32),
          pltpu.VMEM_SHARED((ACC_ROWS, D), jnp.float32),
      ],
  )(pidx)


def _scatter_kernel(t_hbm, pidx_hbm, parts_hbm,
                    rows0_v, rows1_v, sidx_v, didx_v, zbuf, acc_sh,
                    gsem0, gsem1):
  cid = lax.axis_index("c")
  sid = lax.axis_index("s")

  for i in range(ZROWS):
    for q in range(8):
      zbuf[i, pl.ds(q * 16, 16)] = jnp.zeros((16,), jnp.float32)

  for k in range(2):
    rel = cid * 2 + k
    for z in range(ZREP):
      pltpu.sync_copy(zbuf, acc_sh.at[pl.ds(sid * ACC_PER_SUB + z * ZROWS, ZROWS)])
    pltpu.sync_copy(
        pidx_hbm.at[rel, pl.ds(sid * CHUNKS_PER_SUB, CHUNKS_PER_SUB)], didx_v)

    for i in range(CHUNKS_PER_SUB):
      for q in range(8):
        v = didx_v[i, pl.ds(q * 16, 16)]
        src = jnp.minimum(v & 0xFFFF, N_PAD - 1) + rel * N_PAD
        sidx_v[i, pl.ds(q * 16, 16)] = src
        didx_v[i, pl.ds(q * 16, 16)] = v >> 16

    plsc.subcore_barrier()

    rows = (rows0_v, rows1_v)
    sems = (gsem0, gsem1)
    pend = pltpu.async_copy(t_hbm.at[sidx_v.at[0]], rows[0], sems[0])
    for j in range(CHUNKS_PER_SUB):
      pend.wait()
      if j + 1 < CHUNKS_PER_SUB:
        pend = pltpu.async_copy(
            t_hbm.at[sidx_v.at[j + 1]], rows[(j + 1) % 2], sems[(j + 1) % 2])
      pltpu.sync_copy(rows[j % 2], acc_sh.at[didx_v.at[j]], add=True)

    plsc.subcore_barrier()
    pltpu.sync_copy(
        acc_sh.at[pl.ds(sid * OUT_PER_SUB, OUT_PER_SUB)],
        parts_hbm.at[rel, pl.ds(sid * OUT_PER_SUB, OUT_PER_SUB)],
    )
    plsc.subcore_barrier()


def _sc_scatter(t, pidx):
  return pl.kernel(
      _scatter_kernel,
      out_type=jax.ShapeDtypeStruct((R, N_PAD, D), jnp.float32),
      mesh=_mesh(),
      scratch_types=[
          pltpu.VMEM((LANES, D), jnp.float32),
          pltpu.VMEM((LANES, D), jnp.float32),
          pltpu.VMEM((CHUNKS_PER_SUB, LANES), jnp.int32),
          pltpu.VMEM((CHUNKS_PER_SUB, LANES), jnp.int32),
          pltpu.VMEM((ZROWS, D), jnp.float32),
          pltpu.VMEM_SHARED((ACC_ROWS, D), jnp.float32),
          pltpu.SemaphoreType.DMA,
          pltpu.SemaphoreType.DMA,
      ],
  )(t, pidx)


BN = 1024


def _tc_prep_body(degs, x, w, t0, ins, outs):
  for r in range(R):
    od = degs[2 * r, :, 0]
    idg = degs[2 * r + 1, :, 0]
    os_ = lax.rsqrt(jnp.maximum(od, 1.0))
    is_ = lax.rsqrt(jnp.maximum(idg, 1.0))
    outs[r, :] = os_
    ins[r, :] = is_
    t0[r, :, :] = jnp.dot(os_[:, None] * x[...], w[r],
                          preferred_element_type=jnp.float32)


def _tc_prep(degs, x_pad, w0):
  grid = (N_PAD // BN,)
  return pl.pallas_call(
      _tc_prep_body,
      grid=grid,
      in_specs=[
          pl.BlockSpec((2 * R, BN, D), lambda i: (0, i, 0)),
          pl.BlockSpec((BN, D), lambda i: (i, 0)),
          pl.BlockSpec((R, D, D), lambda i: (0, 0, 0)),
      ],
      out_specs=[
          pl.BlockSpec((R, BN, D), lambda i: (0, i, 0)),
          pl.BlockSpec((R, BN), lambda i: (0, i)),
          pl.BlockSpec((R, BN), lambda i: (0, i)),
      ],
      out_shape=[
          jax.ShapeDtypeStruct((R, N_PAD, D), jnp.float32),
          jax.ShapeDtypeStruct((R, N_PAD), jnp.float32),
          jax.ShapeDtypeStruct((R, N_PAD), jnp.float32),
      ],
  )(degs, x_pad, w0)


def _tc_mid_body(parts, ins, outs, wn, b, tn):
  h = jnp.zeros_like(parts[0])
  for r in range(R):
    h = h + jnp.maximum(ins[r, :][:, None] * parts[r] + b[0, r][None, :], 0.0)
  for r in range(R):
    tn[r, :, :] = jnp.dot(outs[r, :][:, None] * h, wn[r],
                          preferred_element_type=jnp.float32)


def _tc_mid(parts, ins, outs, wn, b_l):
  grid = (N_PAD // BN,)
  return pl.pallas_call(
      _tc_mid_body,
      grid=grid,
      in_specs=[
          pl.BlockSpec((R, BN, D), lambda i: (0, i, 0)),
          pl.BlockSpec((R, BN), lambda i: (0, i)),
          pl.BlockSpec((R, BN), lambda i: (0, i)),
          pl.BlockSpec((R, D, D), lambda i: (0, 0, 0)),
          pl.BlockSpec((1, R, D), lambda i: (0, 0, 0)),
      ],
      out_specs=pl.BlockSpec((R, BN, D), lambda i: (0, i, 0)),
      out_shape=jax.ShapeDtypeStruct((R, N_PAD, D), jnp.float32),
  )(parts, ins, outs, wn, b_l)


def _tc_final_body(parts, ins, b, h_out):
  h = b[0].sum(axis=0)[None, :]
  for r in range(R):
    h = h + ins[r, :][:, None] * parts[r]
  h_out[...] = h


def _tc_final(parts, ins, b_l):
  grid = (N_PAD // BN,)
  return pl.pallas_call(
      _tc_final_body,
      grid=grid,
      in_specs=[
          pl.BlockSpec((R, BN, D), lambda i: (0, i, 0)),
          pl.BlockSpec((R, BN), lambda i: (0, i)),
          pl.BlockSpec((1, R, D), lambda i: (0, 0, 0)),
      ],
      out_specs=pl.BlockSpec((BN, D), lambda i: (i, 0)),
      out_shape=jax.ShapeDtypeStruct((N_PAD, D), jnp.float32),
  )(parts, ins, b_l)


def kernel(x, edge_index_r0, edge_index_r1, edge_index_r2, edge_index_r3, Ws, bs):
  edges = [edge_index_r0, edge_index_r1, edge_index_r2, edge_index_r3]

  pad = E_PAD - E
  pidx = []
  for r in range(R):
    src = jnp.concatenate([edges[r][0], jnp.full((pad,), N_PAD, jnp.int32)])
    dst = jnp.concatenate([edges[r][1], jnp.full((pad,), N_PAD, jnp.int32)])
    pidx.append(((dst << 16) | src).reshape(CHUNKS, LANES))
  pidx = jnp.stack(pidx)

  x_pad = jnp.pad(x, ((0, N_PAD - N), (0, 0)))

  degs = _sc_count(pidx)
  t, ins, outs = _tc_prep(degs, x_pad, Ws[0])
  for l in range(L):
    parts = _sc_scatter(t.reshape(R * N_PAD, D), pidx)
    b_l = bs[l][None]
    if l < L - 1:
      t = _tc_mid(parts, ins, outs, Ws[l + 1], b_l)
    else:
      h = _tc_final(parts, ins, b_l)
  return h[:N]


# speedup vs baseline: 3.1977x; 1.0018x over previous
import functools

import jax
import jax.numpy as jnp
from jax import lax
from jax.experimental import pallas as pl
from jax.experimental.pallas import tpu as pltpu
from jax.experimental.pallas import tpu_sc as plsc

N = 10000
D = 128
R = 4
L = 4
E = 80000

NC = 2
NS = 16
LANES = 128

N_PAD = 10240
ACC_ROWS = N_PAD + 128
E_PAD = 81920
CHUNKS = E_PAD // LANES
CHUNKS_PER_SUB = CHUNKS // NS

ACC_PER_SUB = ACC_ROWS // NS
OUT_PER_SUB = N_PAD // NS
ZROWS = 24
ZREP = ACC_PER_SUB // ZROWS

_mesh = functools.partial(
    plsc.VectorSubcoreMesh, core_axis_name="c", subcore_axis_name="s"
)


def _count_kernel(pidx_hbm, degs_hbm, ones_v, sidx_v, didx_v, zbuf, acc_sh,
                  zsem):
  cid = lax.axis_index("c")
  sid = lax.axis_index("s")

  for i in range(LANES):
    for q in range(8):
      zbuf[i, pl.ds(q * 16, 16)] = jnp.zeros((16,), jnp.float32)
      ones_v[i, pl.ds(q * 16, 16)] = jnp.ones((16,), jnp.float32)

  for k in range(2):
    rel = cid * 2 + k
    pltpu.sync_copy(
        pidx_hbm.at[rel, pl.ds(sid * CHUNKS_PER_SUB, CHUNKS_PER_SUB)], didx_v)
    for i in range(CHUNKS_PER_SUB):
      for q in range(8):
        v = didx_v[i, pl.ds(q * 16, 16)]
        sidx_v[i, pl.ds(q * 16, 16)] = v & 0xFFFF
        didx_v[i, pl.ds(q * 16, 16)] = v >> 16

    for d in range(2):
      idx_v = sidx_v if d == 0 else didx_v
      pp = rel * 2 + d
      zd = []
      for z in range(5):
        zd.append(pltpu.async_copy(
            zbuf, acc_sh.at[pl.ds(sid * ACC_PER_SUB + z * LANES, LANES)], zsem))
      zd.append(pltpu.async_copy(
          zbuf.at[pl.ds(0, 8)],
          acc_sh.at[pl.ds(sid * ACC_PER_SUB + 5 * LANES, 8)], zsem))
      for d_ in zd:
        d_.wait()
      plsc.subcore_barrier()

      @pl.loop(0, CHUNKS_PER_SUB)
      def _(j):
        pltpu.sync_copy(ones_v, acc_sh.at[idx_v.at[j]], add=True)

      plsc.subcore_barrier()
      pltpu.sync_copy(
          acc_sh.at[pl.ds(sid * OUT_PER_SUB, OUT_PER_SUB)],
          degs_hbm.at[pp, pl.ds(sid * OUT_PER_SUB, OUT_PER_SUB)],
      )
      plsc.subcore_barrier()


def _sc_count(pidx):
  return pl.kernel(
      _count_kernel,
      out_type=jax.ShapeDtypeStruct((2 * R, N_PAD, D), jnp.float32),
      mesh=_mesh(),
      scratch_types=[
          pltpu.VMEM((LANES, D), jnp.float32),
          pltpu.VMEM((CHUNKS_PER_SUB, LANES), jnp.int32),
          pltpu.VMEM((CHUNKS_PER_SUB, LANES), jnp.int32),
          pltpu.VMEM((LANES, D), jnp.float32),
          pltpu.VMEM_SHARED((ACC_ROWS, D), jnp.float32),
          pltpu.SemaphoreType.DMA,
      ],
  )(pidx)


def _scatter_kernel(t_hbm, pidx_hbm, parts_hbm,
                    rows0_v, rows1_v, sidx_v, didx_v, acc_sh,
                    gsem0, gsem1, zsem):
  cid = lax.axis_index("c")
  sid = lax.axis_index("s")

  for k in range(2):
    rel = cid * 2 + k
    for i in range(LANES):
      for q in range(8):
        rows0_v[i, pl.ds(q * 16, 16)] = jnp.zeros((16,), jnp.float32)
    zd = []
    for z in range(5):
      zd.append(pltpu.async_copy(
          rows0_v, acc_sh.at[pl.ds(sid * ACC_PER_SUB + z * LANES, LANES)], zsem))
    zd.append(pltpu.async_copy(
        rows0_v.at[pl.ds(0, 8)],
        acc_sh.at[pl.ds(sid * ACC_PER_SUB + 5 * LANES, 8)], zsem))
    for d_ in zd:
      d_.wait()
    pltpu.sync_copy(
        pidx_hbm.at[rel, pl.ds(sid * CHUNKS_PER_SUB, CHUNKS_PER_SUB)], didx_v)

    for i in range(CHUNKS_PER_SUB):
      for q in range(8):
        v = didx_v[i, pl.ds(q * 16, 16)]
        src = jnp.minimum(v & 0xFFFF, N_PAD - 1) + rel * N_PAD
        sidx_v[i, pl.ds(q * 16, 16)] = src
        didx_v[i, pl.ds(q * 16, 16)] = v >> 16

    plsc.subcore_barrier()

    rows = (rows0_v, rows1_v)
    sems = (gsem0, gsem1)
    pend = pltpu.async_copy(t_hbm.at[sidx_v.at[0]], rows[0], sems[0])
    for j in range(CHUNKS_PER_SUB):
      pend.wait()
      if j + 1 < CHUNKS_PER_SUB:
        pend = pltpu.async_copy(
            t_hbm.at[sidx_v.at[j + 1]], rows[(j + 1) % 2], sems[(j + 1) % 2])
      pltpu.sync_copy(rows[j % 2], acc_sh.at[didx_v.at[j]], add=True)

    plsc.subcore_barrier()
    pltpu.sync_copy(
        acc_sh.at[pl.ds(sid * OUT_PER_SUB, OUT_PER_SUB)],
        parts_hbm.at[rel, pl.ds(sid * OUT_PER_SUB, OUT_PER_SUB)],
    )
    plsc.subcore_barrier()


def _sc_scatter(t, pidx):
  return pl.kernel(
      _scatter_kernel,
      out_type=jax.ShapeDtypeStruct((R, N_PAD, D), jnp.float32),
      mesh=_mesh(),
      scratch_types=[
          pltpu.VMEM((LANES, D), jnp.float32),
          pltpu.VMEM((LANES, D), jnp.float32),
          pltpu.VMEM((CHUNKS_PER_SUB, LANES), jnp.int32),
          pltpu.VMEM((CHUNKS_PER_SUB, LANES), jnp.int32),
          pltpu.VMEM_SHARED((ACC_ROWS, D), jnp.float32),
          pltpu.SemaphoreType.DMA,
          pltpu.SemaphoreType.DMA,
          pltpu.SemaphoreType.DMA,
      ],
  )(t, pidx)


BN = 1024


def _tc_prep_body(degs, x, w, t0, ins, outs):
  for r in range(R):
    od = degs[2 * r, :, 0]
    idg = degs[2 * r + 1, :, 0]
    os_ = lax.rsqrt(jnp.maximum(od, 1.0))
    is_ = lax.rsqrt(jnp.maximum(idg, 1.0))
    outs[r, :] = os_
    ins[r, :] = is_
    t0[r, :, :] = jnp.dot(os_[:, None] * x[...], w[r],
                          preferred_element_type=jnp.float32)


def _tc_prep(degs, x_pad, w0):
  grid = (N_PAD // BN,)
  return pl.pallas_call(
      _tc_prep_body,
      grid=grid,
      in_specs=[
          pl.BlockSpec((2 * R, BN, D), lambda i: (0, i, 0)),
          pl.BlockSpec((BN, D), lambda i: (i, 0)),
          pl.BlockSpec((R, D, D), lambda i: (0, 0, 0)),
      ],
      out_specs=[
          pl.BlockSpec((R, BN, D), lambda i: (0, i, 0)),
          pl.BlockSpec((R, BN), lambda i: (0, i)),
          pl.BlockSpec((R, BN), lambda i: (0, i)),
      ],
      out_shape=[
          jax.ShapeDtypeStruct((R, N_PAD, D), jnp.float32),
          jax.ShapeDtypeStruct((R, N_PAD), jnp.float32),
          jax.ShapeDtypeStruct((R, N_PAD), jnp.float32),
      ],
  )(degs, x_pad, w0)


def _tc_mid_body(parts, ins, outs, wn, b, tn):
  h = jnp.zeros_like(parts[0])
  for r in range(R):
    h = h + jnp.maximum(ins[r, :][:, None] * parts[r] + b[0, r][None, :], 0.0)
  for r in range(R):
    tn[r, :, :] = jnp.dot(outs[r, :][:, None] * h, wn[r],
                          preferred_element_type=jnp.float32)


def _tc_mid(parts, ins, outs, wn, b_l):
  grid = (N_PAD // BN,)
  return pl.pallas_call(
      _tc_mid_body,
      grid=grid,
      in_specs=[
          pl.BlockSpec((R, BN, D), lambda i: (0, i, 0)),
          pl.BlockSpec((R, BN), lambda i: (0, i)),
          pl.BlockSpec((R, BN), lambda i: (0, i)),
          pl.BlockSpec((R, D, D), lambda i: (0, 0, 0)),
          pl.BlockSpec((1, R, D), lambda i: (0, 0, 0)),
      ],
      out_specs=pl.BlockSpec((R, BN, D), lambda i: (0, i, 0)),
      out_shape=jax.ShapeDtypeStruct((R, N_PAD, D), jnp.float32),
  )(parts, ins, outs, wn, b_l)


def _tc_final_body(parts, ins, b, h_out):
  h = b[0].sum(axis=0)[None, :]
  for r in range(R):
    h = h + ins[r, :][:, None] * parts[r]
  h_out[...] = h


def _tc_final(parts, ins, b_l):
  grid = (N_PAD // BN,)
  return pl.pallas_call(
      _tc_final_body,
      grid=grid,
      in_specs=[
          pl.BlockSpec((R, BN, D), lambda i: (0, i, 0)),
          pl.BlockSpec((R, BN), lambda i: (0, i)),
          pl.BlockSpec((1, R, D), lambda i: (0, 0, 0)),
      ],
      out_specs=pl.BlockSpec((BN, D), lambda i: (i, 0)),
      out_shape=jax.ShapeDtypeStruct((N_PAD, D), jnp.float32),
  )(parts, ins, b_l)


def kernel(x, edge_index_r0, edge_index_r1, edge_index_r2, edge_index_r3, Ws, bs):
  edges = [edge_index_r0, edge_index_r1, edge_index_r2, edge_index_r3]

  pad = E_PAD - E
  pidx = []
  for r in range(R):
    src = jnp.concatenate([edges[r][0], jnp.full((pad,), N_PAD, jnp.int32)])
    dst = jnp.concatenate([edges[r][1], jnp.full((pad,), N_PAD, jnp.int32)])
    pidx.append(((dst << 16) | src).reshape(CHUNKS, LANES))
  pidx = jnp.stack(pidx)

  x_pad = jnp.pad(x, ((0, N_PAD - N), (0, 0)))

  degs = _sc_count(pidx)
  t, ins, outs = _tc_prep(degs, x_pad, Ws[0])
  for l in range(L):
    parts = _sc_scatter(t.reshape(R * N_PAD, D), pidx)
    b_l = bs[l][None]
    if l < L - 1:
      t = _tc_mid(parts, ins, outs, Ws[l + 1], b_l)
    else:
      h = _tc_final(parts, ins, b_l)
  return h[:N]
